# baseline (speedup 1.0000x reference)
import functools

import jax
import jax.numpy as jnp
from jax import lax
from jax.experimental import pallas as pl
from jax.experimental.pallas import tpu as pltpu
from jax.experimental.pallas import tpu_sc as plsc

POOL = 1024
PLEN = 8
DIM = 768
K = 5
BATCH = 1024

NC = 2
NS = 16
LANES = 16
NEG = -3.0e38


def _tc_body(q_ref, k_ref, sim_ref, avg_ref):
    q = q_ref[...]
    k = k_ref[...]
    qn = q / jnp.maximum(jnp.sqrt(jnp.sum(q * q, axis=1, keepdims=True)), 1e-12)
    kn = k / jnp.maximum(jnp.sqrt(jnp.sum(k * k, axis=1, keepdims=True)), 1e-12)
    sim = lax.dot_general(qn, kn, (((1,), (1,)), ((), ())),
                          preferred_element_type=jnp.float32)
    sim_ref[...] = sim
    avg_ref[...] = jnp.mean(sim, axis=0, keepdims=True)


_tc_call = pl.pallas_call(
    _tc_body,
    out_shape=(
        jax.ShapeDtypeStruct((BATCH, POOL), jnp.float32),
        jax.ShapeDtypeStruct((1, POOL), jnp.float32),
    ),
)


def _sc_body(avg_hbm, pool_hbm, out_hbm, avg_v, idx_v, rows_v, gsem):
    cid = lax.axis_index("c")
    sid = lax.axis_index("s")
    wid = sid * NC + cid

    @pl.when(wid == 0)
    def _():
        pltpu.sync_copy(avg_hbm, avg_v)

        iota = lax.iota(jnp.int32, LANES)

        def _xlane(v, perm):
            return v.at[perm].get(mode="promise_in_bounds")

        def _bcast_max(v):
            for s in (1, 2, 4, 8):
                v = jnp.maximum(v, _xlane(v, iota ^ s))
            return v

        def _bcast_min(v):
            for s in (1, 2, 4, 8):
                v = jnp.minimum(v, _xlane(v, iota ^ s))
            return v

        idxs = jnp.zeros((LANES,), jnp.int32)
        chosen_list = []
        for r in range(K):
            def chunk_body(c, carry, _chosen=tuple(chosen_list)):
                bestv, besti = carry
                v = avg_v[pl.ds(c * LANES, LANES)]
                i = c * LANES + iota
                m = v > bestv
                for cv in _chosen:
                    m = m & (i != cv)
                return (jnp.where(m, v, bestv), jnp.where(m, i, besti))

            bestv, besti = lax.fori_loop(
                0, POOL // LANES, chunk_body,
                (jnp.full((LANES,), NEG, jnp.float32),
                 jnp.zeros((LANES,), jnp.int32)))
            gmax = _bcast_max(bestv)
            chosen_vec = _bcast_min(
                jnp.where(bestv == gmax, besti, jnp.int32(2 ** 30)))
            idxs = jnp.where(iota == r, chosen_vec, idxs)
            chosen_list.append(chosen_vec)

        idx_v[...] = idxs
        pltpu.async_copy(pool_hbm.at[idx_v.at[pl.ds(0, 8)]], rows_v, gsem).wait()
        pltpu.sync_copy(rows_v.at[pl.ds(0, K)], out_hbm)


_sc_call = functools.partial(
    pl.kernel,
    out_type=jax.ShapeDtypeStruct((K, PLEN, DIM), jnp.float32),
    mesh=plsc.VectorSubcoreMesh(core_axis_name="c", subcore_axis_name="s"),
    scratch_types=[
        pltpu.VMEM((POOL,), jnp.float32),
        pltpu.VMEM((LANES,), jnp.int32),
        pltpu.VMEM((8, PLEN, DIM), jnp.float32),
        pltpu.SemaphoreType.DMA,
    ],
)(_sc_body)

BB = 32


def _bc_body(sel_ref, out_ref):
    out_ref[...] = jnp.broadcast_to(sel_ref[...][None], (BB, K, PLEN, DIM))


_bc_call = pl.pallas_call(
    _bc_body,
    grid=(BATCH // BB,),
    in_specs=[pl.BlockSpec((K, PLEN, DIM), lambda b: (0, 0, 0))],
    out_specs=pl.BlockSpec((BB, K, PLEN, DIM), lambda b: (b, 0, 0, 0)),
    out_shape=jax.ShapeDtypeStruct((BATCH, K, PLEN, DIM), jnp.float32),
)


def kernel(query, prompt_pool, prompt_key):
    sim, avg = _tc_call(query, prompt_key)
    sel_small = _sc_call(avg.reshape(POOL), prompt_pool)
    sel = _bc_call(sel_small)
    selected_prompts = sel.reshape(BATCH, K * PLEN, DIM)
    prompt_mask = jnp.ones((BATCH, K * PLEN), dtype=bool)
    return selected_prompts, prompt_mask, sim
